# baseline (speedup 1.0000x reference)
import functools

import jax
import jax.numpy as jnp
from jax import lax
from jax.experimental import pallas as pl
from jax.experimental.pallas import tpu as pltpu
from jax.experimental.pallas import tpu_sc as plsc

N = 16777216
NBINS = 256
COVERAGE = 0.99
NC = 2
NS = 16
L = 16
NW = NC * NS
PER_SUB = N // NW
CHUNK = 32768
NCHUNK = PER_SUB // CHUNK
VREGS = CHUNK // L
UNROLL = 8

_mesh = plsc.VectorSubcoreMesh(core_axis_name="c", subcore_axis_name="s")
_f32 = jnp.float32


def _fold_scalar(vec, op):
    acc = vec[0]
    for i in range(1, L):
        acc = op(acc, vec[i])
    return acc


@functools.partial(
    pl.kernel,
    out_type=jax.ShapeDtypeStruct((2 * L,), _f32),
    mesh=_mesh,
    compiler_params=pltpu.CompilerParams(needs_layout_passes=False),
    scratch_types=[
        pltpu.VMEM((CHUNK,), _f32),
        pltpu.VMEM((CHUNK,), _f32),
        pltpu.VMEM(((NBINS + 1) * L,), _f32),
        pltpu.VMEM((NBINS,), _f32),
        pltpu.VMEM((NW * L,), _f32),
        pltpu.VMEM((NW * L,), _f32),
        pltpu.VMEM((NW * NBINS,), _f32),
        pltpu.VMEM((2 * L,), _f32),
        pltpu.VMEM((L,), _f32),
        pltpu.VMEM((NW * L,), _f32),
        pltpu.SemaphoreType.DMA,
        pltpu.SemaphoreType.DMA,
    ],
)
def _fused_k(x_hbm, flags1_hbm, flags2_hbm, mins_hbm, maxs_hbm, hist_hbm,
             out_hbm, buf0, buf1, table, hstage, minsv, maxsv, histv, stag,
             fones, flv, sem0, sem1):
    wid = lax.axis_index("s") * NC + lax.axis_index("c")
    base = wid * PER_SUB
    bufs = (buf0, buf1)
    sems = (sem0, sem1)
    zeros = jnp.zeros((L,), _f32)
    ones = jnp.full((L,), 1.0, dtype=_f32)
    lane_i = lax.iota(jnp.int32, L)

    cps = [None, None]
    cps[0] = pltpu.async_copy(x_hbm.at[pl.ds(base, CHUNK)], buf0, sem0)

    big = jnp.full((L,), 3.4e38, dtype=_f32)
    NACC = 4
    mns = [big] * NACC
    mxs = [-big] * NACC
    for k in range(NCHUNK):
        nxt = k + 1
        if nxt < NCHUNK:
            cps[nxt % 2] = pltpu.async_copy(
                x_hbm.at[pl.ds(base + nxt * CHUNK, CHUNK)], bufs[nxt % 2],
                sems[nxt % 2])
        cps[k % 2].wait()
        buf = bufs[k % 2]

        def body(i, carry, buf=buf):
            acc = list(carry)
            for u in range(UNROLL):
                x = buf[pl.ds((i * UNROLL + u) * L, L)]
                a = u % NACC
                acc[a] = jnp.minimum(acc[a], x)
                acc[NACC + a] = jnp.maximum(acc[NACC + a], x)
            return tuple(acc)

        res = lax.fori_loop(0, VREGS // UNROLL, body, tuple(mns + mxs))
        mns = list(res[:NACC])
        mxs = list(res[NACC:])

    mn = mns[0]
    mx = mxs[0]
    for a in range(1, NACC):
        mn = jnp.minimum(mn, mns[a])
        mx = jnp.maximum(mx, mxs[a])
    stag[pl.ds(0, L)] = mn
    stag[pl.ds(L, L)] = mx
    pltpu.sync_copy(stag.at[pl.ds(0, L)], mins_hbm.at[pl.ds(wid * L, L)])
    pltpu.sync_copy(stag.at[pl.ds(L, L)], maxs_hbm.at[pl.ds(wid * L, L)])
    fones[...] = ones
    pltpu.sync_copy(fones, flags1_hbm.at[pl.ds(wid * L, L)])

    cps[0] = pltpu.async_copy(x_hbm.at[pl.ds(base, CHUNK)], buf0, sem0)
    cps[1] = pltpu.async_copy(x_hbm.at[pl.ds(base + CHUNK, CHUNK)], buf1,
                              sem1)
    for b in range(NBINS + 1):
        table[pl.ds(b * L, L)] = zeros

    full = _f32(NW)

    def _poll_cond(c):
        return c < full

    def _poll_flags1(c):
        pltpu.sync_copy(flags1_hbm, flv)
        s = flv[pl.ds(0, L)]
        for r in range(1, NW):
            s = s + flv[pl.ds(r * L, L)]
        return s[0]

    lax.while_loop(_poll_cond, _poll_flags1, _f32(0.0))
    pltpu.sync_copy(mins_hbm, minsv)
    pltpu.sync_copy(maxs_hbm, maxsv)

    mnv = minsv[pl.ds(0, L)]
    mxv = maxsv[pl.ds(0, L)]
    for r in range(1, NW):
        mnv = jnp.minimum(mnv, minsv[pl.ds(r * L, L)])
        mxv = jnp.maximum(mxv, maxsv[pl.ds(r * L, L)])
    tmin = _fold_scalar(mnv, jnp.minimum)
    tmax = _fold_scalar(mxv, jnp.maximum)
    tmin_v = zeros + tmin
    width = (tmax - tmin) * _f32(1.0 / NBINS)
    width_v = zeros + width
    seed_i = jnp.full((L,), 0x7EF311C3, jnp.int32) - plsc.bitcast(
        width_v, jnp.int32)
    inv_v = plsc.bitcast(seed_i, _f32)
    two_v = jnp.full((L,), 2.0, dtype=_f32)
    for _ in range(3):
        inv_v = inv_v * (two_v - width_v * inv_v)
    inv16_v = inv_v * jnp.full((L,), 16.0, dtype=_f32)

    for k in range(NCHUNK):
        cps[k % 2].wait()
        buf = bufs[k % 2]

        @plsc.parallel_loop(0, VREGS, unroll=UNROLL)
        def body(i, buf=buf):
            x = buf[pl.ds(i * L, L)]
            t16 = (x - tmin_v) * inv16_v
            plsc.addupdate_scatter(table, [t16.astype(jnp.int32)], ones)

        nxt = k + 2
        if nxt < NCHUNK:
            cps[nxt % 2] = pltpu.async_copy(
                x_hbm.at[pl.ds(base + nxt * CHUNK, CHUNK)], bufs[nxt % 2],
                sems[nxt % 2])

    perms = [plsc.bitcast((lax.iota(jnp.int32, L) + (1 << p)) & (L - 1),
                          jnp.int32) for p in range(3, -1, -1)]
    mask0 = lane_i == 0
    for b in range(NBINS):
        v = table[pl.ds(b * L, L)]
        if b == NBINS - 1:
            v = v + table[pl.ds(NBINS * L, L)]
        for perm in perms:
            v = v + v.at[perm].get(mode="promise_in_bounds")
        plsc.store_scatter(hstage, [jnp.zeros((L,), jnp.int32) + b], v,
                           mask=mask0)
    pltpu.sync_copy(hstage, hist_hbm.at[pl.ds(wid * NBINS, NBINS)])
    pltpu.sync_copy(fones, flags2_hbm.at[pl.ds(wid * L, L)])

    @pl.when(wid == 0)
    def _():
        def _poll_flags2(c):
            pltpu.sync_copy(flags2_hbm, flv)
            s = flv[pl.ds(0, L)]
            for r in range(1, NW):
                s = s + flv[pl.ds(r * L, L)]
            return s[0]

        lax.while_loop(_poll_cond, _poll_flags2, _f32(0.0))
        pltpu.sync_copy(hist_hbm, histv)

        total = jnp.float32(N)
        t_lo = total * jnp.float32((1.0 - COVERAGE) / 2.0)
        t_hi = total * jnp.float32((1.0 + COVERAGE) / 2.0)

        zero = jnp.zeros((), _f32)
        one = jnp.ones((), _f32)
        cum = zero
        lo_idx = zero
        hi_idx = zero
        for c in range(NBINS // L):
            acc = histv[pl.ds(c * L, L)]
            for r in range(1, NW):
                acc = acc + histv[pl.ds(r * NBINS + c * L, L)]
            for i in range(L):
                cum = cum + acc[i]
                lo_idx = lo_idx + jnp.where(cum <= t_lo, one, zero)
                hi_idx = hi_idx + jnp.where(cum <= t_hi, one, zero)

        min_value = tmin + lo_idx * width
        max_value = tmin + hi_idx * width
        stag[pl.ds(0, L)] = zeros + min_value
        stag[pl.ds(L, L)] = zeros + max_value
        pltpu.sync_copy(stag, out_hbm)


def kernel(tensor):
    flags1 = jax.new_ref(jnp.zeros((NW * L,), _f32))
    flags2 = jax.new_ref(jnp.zeros((NW * L,), _f32))
    mins = jax.new_ref(jnp.zeros((NW * L,), _f32))
    maxs = jax.new_ref(jnp.zeros((NW * L,), _f32))
    hist = jax.new_ref(jnp.zeros((NW * NBINS,), _f32))
    out = _fused_k(tensor, flags1, flags2, mins, maxs, hist)
    return (out[0], out[L])


# speedup vs baseline: 1.1499x; 1.1499x over previous
import functools

import jax
import jax.numpy as jnp
from jax import lax
from jax.experimental import pallas as pl
from jax.experimental.pallas import tpu as pltpu
from jax.experimental.pallas import tpu_sc as plsc

N = 16777216
NBINS = 256
COVERAGE = 0.99
NC = 2
NS = 16
L = 16
NW = NC * NS
PER_SUB = N // NW
CHUNK = 32768
NCHUNK = PER_SUB // CHUNK
VREGS = CHUNK // L
UNROLL = 8

_mesh = plsc.VectorSubcoreMesh(core_axis_name="c", subcore_axis_name="s")
_f32 = jnp.float32


def _fold_scalar(vec, op):
    acc = vec[0]
    for i in range(1, L):
        acc = op(acc, vec[i])
    return acc


@functools.partial(
    pl.kernel,
    out_type=jax.ShapeDtypeStruct((2 * L,), _f32),
    mesh=_mesh,
    compiler_params=pltpu.CompilerParams(needs_layout_passes=False),
    scratch_types=[
        pltpu.VMEM((CHUNK,), _f32),
        pltpu.VMEM((CHUNK,), _f32),
        pltpu.VMEM(((NBINS + 1) * L,), _f32),
        pltpu.VMEM((NBINS,), _f32),
        pltpu.VMEM((NW * L,), _f32),
        pltpu.VMEM((NW * L,), _f32),
        pltpu.VMEM((NW * NBINS,), _f32),
        pltpu.VMEM((2 * L,), _f32),
        pltpu.VMEM((L,), _f32),
        pltpu.VMEM((NW * L,), _f32),
        pltpu.SemaphoreType.DMA,
        pltpu.SemaphoreType.DMA,
    ],
)
def _fused_k(x_hbm, flags1_hbm, flags2_hbm, mins_hbm, maxs_hbm, hist_hbm,
             out_hbm, buf0, buf1, table, hstage, minsv, maxsv, histv, stag,
             fones, flv, sem0, sem1):
    wid = lax.axis_index("s") * NC + lax.axis_index("c")
    base = wid * PER_SUB
    bufs = (buf0, buf1)
    sems = (sem0, sem1)
    zeros = jnp.zeros((L,), _f32)
    ones = jnp.full((L,), 1.0, dtype=_f32)
    lane_i = lax.iota(jnp.int32, L)

    cps = [None, None]
    cps[0] = pltpu.async_copy(x_hbm.at[pl.ds(base, CHUNK)], buf0, sem0)

    big = jnp.full((L,), 3.4e38, dtype=_f32)
    NACC = 4
    mns = [big] * NACC
    mxs = [-big] * NACC
    for k in range(NCHUNK):
        nxt = k + 1
        if nxt < NCHUNK:
            cps[nxt % 2] = pltpu.async_copy(
                x_hbm.at[pl.ds(base + nxt * CHUNK, CHUNK)], bufs[nxt % 2],
                sems[nxt % 2])
        cps[k % 2].wait()
        buf = bufs[k % 2]

        def body(i, carry, buf=buf):
            acc = list(carry)
            for u in range(UNROLL):
                x = buf[pl.ds((i * UNROLL + u) * L, L)]
                a = u % NACC
                acc[a] = jnp.minimum(acc[a], x)
                acc[NACC + a] = jnp.maximum(acc[NACC + a], x)
            return tuple(acc)

        res = lax.fori_loop(0, VREGS // UNROLL, body, tuple(mns + mxs))
        mns = list(res[:NACC])
        mxs = list(res[NACC:])

    mn = mns[0]
    mx = mxs[0]
    for a in range(1, NACC):
        mn = jnp.minimum(mn, mns[a])
        mx = jnp.maximum(mx, mxs[a])
    stag[pl.ds(0, L)] = mn
    stag[pl.ds(L, L)] = mx
    pltpu.sync_copy(stag.at[pl.ds(0, L)], mins_hbm.at[pl.ds(wid * L, L)])
    pltpu.sync_copy(stag.at[pl.ds(L, L)], maxs_hbm.at[pl.ds(wid * L, L)])
    fones[...] = ones
    pltpu.sync_copy(fones, flags1_hbm.at[pl.ds(wid * L, L)])

    cps[0] = pltpu.async_copy(x_hbm.at[pl.ds(base, CHUNK)], buf0, sem0)
    cps[1] = pltpu.async_copy(x_hbm.at[pl.ds(base + CHUNK, CHUNK)], buf1,
                              sem1)
    for b in range(NBINS + 1):
        table[pl.ds(b * L, L)] = zeros

    full = _f32(NW)

    def _poll_cond(c):
        return c < full

    def _poll_flags1(c):
        pltpu.sync_copy(flags1_hbm, flv)
        s = flv[pl.ds(0, L)]
        for r in range(1, NW):
            s = s + flv[pl.ds(r * L, L)]
        return s[0]

    lax.while_loop(_poll_cond, _poll_flags1, _f32(0.0))
    pltpu.sync_copy(mins_hbm, minsv)
    pltpu.sync_copy(maxs_hbm, maxsv)

    mnv = minsv[pl.ds(0, L)]
    mxv = maxsv[pl.ds(0, L)]
    for r in range(1, NW):
        mnv = jnp.minimum(mnv, minsv[pl.ds(r * L, L)])
        mxv = jnp.maximum(mxv, maxsv[pl.ds(r * L, L)])
    tmin = _fold_scalar(mnv, jnp.minimum)
    tmax = _fold_scalar(mxv, jnp.maximum)
    tmin_v = zeros + tmin
    width = (tmax - tmin) * _f32(1.0 / NBINS)
    width_v = zeros + width
    seed_i = jnp.full((L,), 0x7EF311C3, jnp.int32) - plsc.bitcast(
        width_v, jnp.int32)
    inv_v = plsc.bitcast(seed_i, _f32)
    two_v = jnp.full((L,), 2.0, dtype=_f32)
    for _ in range(3):
        inv_v = inv_v * (two_v - width_v * inv_v)

    for k in range(NCHUNK):
        cps[k % 2].wait()
        buf = bufs[k % 2]

        @plsc.parallel_loop(0, VREGS, unroll=UNROLL)
        def body(i, buf=buf):
            x = buf[pl.ds(i * L, L)]
            t = (x - tmin_v) * inv_v
            idx = (t.astype(jnp.int32) << 4) | lane_i
            plsc.addupdate_scatter(table, [idx], ones)

        nxt = k + 2
        if nxt < NCHUNK:
            cps[nxt % 2] = pltpu.async_copy(
                x_hbm.at[pl.ds(base + nxt * CHUNK, CHUNK)], bufs[nxt % 2],
                sems[nxt % 2])

    perms = [plsc.bitcast((lax.iota(jnp.int32, L) + (1 << p)) & (L - 1),
                          jnp.int32) for p in range(3, -1, -1)]
    mask0 = lane_i == 0
    for b in range(NBINS):
        v = table[pl.ds(b * L, L)]
        if b == NBINS - 1:
            v = v + table[pl.ds(NBINS * L, L)]
        for perm in perms:
            v = v + v.at[perm].get(mode="promise_in_bounds")
        plsc.store_scatter(hstage, [jnp.zeros((L,), jnp.int32) + b], v,
                           mask=mask0)
    pltpu.sync_copy(hstage, hist_hbm.at[pl.ds(wid * NBINS, NBINS)])
    pltpu.sync_copy(fones, flags2_hbm.at[pl.ds(wid * L, L)])

    @pl.when(wid == 0)
    def _():
        def _poll_flags2(c):
            pltpu.sync_copy(flags2_hbm, flv)
            s = flv[pl.ds(0, L)]
            for r in range(1, NW):
                s = s + flv[pl.ds(r * L, L)]
            return s[0]

        lax.while_loop(_poll_cond, _poll_flags2, _f32(0.0))
        pltpu.sync_copy(hist_hbm, histv)

        total = jnp.float32(N)
        t_lo = total * jnp.float32((1.0 - COVERAGE) / 2.0)
        t_hi = total * jnp.float32((1.0 + COVERAGE) / 2.0)

        zero = jnp.zeros((), _f32)
        one = jnp.ones((), _f32)
        cum = zero
        lo_idx = zero
        hi_idx = zero
        for c in range(NBINS // L):
            acc = histv[pl.ds(c * L, L)]
            for r in range(1, NW):
                acc = acc + histv[pl.ds(r * NBINS + c * L, L)]
            for i in range(L):
                cum = cum + acc[i]
                lo_idx = lo_idx + jnp.where(cum <= t_lo, one, zero)
                hi_idx = hi_idx + jnp.where(cum <= t_hi, one, zero)

        min_value = tmin + lo_idx * width
        max_value = tmin + hi_idx * width
        stag[pl.ds(0, L)] = zeros + min_value
        stag[pl.ds(L, L)] = zeros + max_value
        pltpu.sync_copy(stag, out_hbm)


def kernel(tensor):
    flags1 = jax.new_ref(jnp.zeros((NW * L,), _f32))
    flags2 = jax.new_ref(jnp.zeros((NW * L,), _f32))
    mins = jax.new_ref(jnp.zeros((NW * L,), _f32))
    maxs = jax.new_ref(jnp.zeros((NW * L,), _f32))
    hist = jax.new_ref(jnp.zeros((NW * NBINS,), _f32))
    out = _fused_k(tensor, flags1, flags2, mins, maxs, hist)
    return (out[0], out[L])
